# Initial kernel scaffold; baseline (speedup 1.0000x reference)
import jax
import jax.numpy as jnp
from jax.experimental import pallas as pl


def kernel(x, edge_index, W0, W1, W2):
    raise NotImplementedError("write your pallas kernel here")


# speedup vs baseline: 9.4936x; 9.4936x over previous
import functools

import jax
import jax.numpy as jnp
from jax import lax
from jax.experimental import pallas as pl
from jax.experimental.pallas import tpu as pltpu
from jax.experimental.pallas import tpu_sc as plsc

N_NODES = 10000
N_EDGES = 320000
DIM = 128

NC, NS = 2, 16
NW = NC * NS
CH = 128
NPAD = 10240
EPT = 79 * CH
EPAD = NW * EPT
ROWS_PER_TILE = NPAD // NS

_mesh = plsc.VectorSubcoreMesh(
    core_axis_name="c", subcore_axis_name="s", num_cores=NC, num_subcores=NS
)


@functools.partial(
    pl.kernel,
    out_type=jax.ShapeDtypeStruct((NC, NPAD), jnp.float32),
    mesh=_mesh,
    scratch_types=[
        pltpu.VMEM_SHARED((NPAD,), jnp.float32),
        pltpu.VMEM((CH,), jnp.int32),
        pltpu.VMEM((CH,), jnp.float32),
    ],
)
def _degree_sc(dst_hbm, ones_hbm, zeros_hbm, out_hbm, hist_sp, idx_v, ones_v):
    c = lax.axis_index("c")
    s = lax.axis_index("s")
    w = s * NC + c
    pltpu.sync_copy(zeros_hbm, hist_sp.at[pl.ds(s * ROWS_PER_TILE, ROWS_PER_TILE)])
    pltpu.sync_copy(ones_hbm, ones_v)
    plsc.subcore_barrier()

    base = w * EPT

    def chunk(i, carry):
        pltpu.sync_copy(dst_hbm.at[pl.ds(base + i * CH, CH)], idx_v)
        pltpu.sync_copy(ones_v, hist_sp.at[idx_v], add=True)
        return carry

    lax.fori_loop(0, EPT // CH, chunk, 0)
    plsc.subcore_barrier()
    pltpu.sync_copy(
        hist_sp.at[pl.ds(s * ROWS_PER_TILE, ROWS_PER_TILE)],
        out_hbm.at[c, pl.ds(s * ROWS_PER_TILE, ROWS_PER_TILE)],
    )


@functools.partial(
    pl.kernel,
    out_type=jax.ShapeDtypeStruct((NC, NPAD, DIM), jnp.float32),
    mesh=_mesh,
    scratch_types=[
        pltpu.VMEM_SHARED((NPAD, DIM), jnp.float32),
        pltpu.VMEM((CH,), jnp.int32),
        pltpu.VMEM((CH,), jnp.int32),
        pltpu.VMEM((CH, DIM), jnp.float32),
        pltpu.SemaphoreType.DMA,
    ],
)
def _scatter_sc(g_hbm, src_hbm, dst_hbm, zeros_hbm, out_hbm,
                accum, sidx, didx, rows, sem):
    c = lax.axis_index("c")
    s = lax.axis_index("s")
    w = s * NC + c
    for k in range(ROWS_PER_TILE // 64):
        pltpu.sync_copy(zeros_hbm, accum.at[pl.ds(s * ROWS_PER_TILE + k * 64, 64)])
    plsc.subcore_barrier()

    base = w * EPT

    def chunk(i, carry):
        off = base + i * CH
        pltpu.sync_copy(src_hbm.at[pl.ds(off, CH)], sidx)
        pltpu.sync_copy(dst_hbm.at[pl.ds(off, CH)], didx)
        pltpu.async_copy(g_hbm.at[sidx], rows, sem).wait()
        pltpu.sync_copy(rows, accum.at[didx], add=True)
        return carry

    lax.fori_loop(0, EPT // CH, chunk, 0)
    plsc.subcore_barrier()
    pltpu.sync_copy(
        accum.at[pl.ds(s * ROWS_PER_TILE, ROWS_PER_TILE)],
        out_hbm.at[c, pl.ds(s * ROWS_PER_TILE, ROWS_PER_TILE)],
    )


_BLK = 1024
_GRID = NPAD // _BLK


def _row_spec():
    return pl.BlockSpec((_BLK, DIM), lambda i: (i, 0))


def _col_spec():
    return pl.BlockSpec((_BLK, 1), lambda i: (i, 0))


def _w_spec():
    return pl.BlockSpec((DIM, DIM), lambda i: (0, 0))


def _first_body(x_ref, w_ref, h0_ref, h1_ref, g_ref, dinv_ref):
    dinv = lax.rsqrt(1.0 + h0_ref[...] + h1_ref[...])
    dinv_ref[...] = dinv
    g_ref[...] = jnp.dot(x_ref[...], w_ref[...],
                         preferred_element_type=jnp.float32) * dinv


_first_tc = pl.pallas_call(
    _first_body,
    grid=(_GRID,),
    in_specs=[_row_spec(), _w_spec(), _col_spec(), _col_spec()],
    out_specs=[_row_spec(), _col_spec()],
    out_shape=[
        jax.ShapeDtypeStruct((NPAD, DIM), jnp.float32),
        jax.ShapeDtypeStruct((NPAD, 1), jnp.float32),
    ],
)


def _mid_body(p0_ref, p1_ref, g_ref, dinv_ref, w_ref, o_ref):
    dinv = dinv_ref[...]
    h = jnp.maximum((p0_ref[...] + p1_ref[...] + g_ref[...]) * dinv, 0.0)
    o_ref[...] = jnp.dot(h, w_ref[...],
                         preferred_element_type=jnp.float32) * dinv


_mid_tc = pl.pallas_call(
    _mid_body,
    grid=(_GRID,),
    in_specs=[_row_spec(), _row_spec(), _row_spec(), _col_spec(), _w_spec()],
    out_specs=_row_spec(),
    out_shape=jax.ShapeDtypeStruct((NPAD, DIM), jnp.float32),
)


def _last_body(p0_ref, p1_ref, g_ref, dinv_ref, o_ref):
    h = jnp.maximum((p0_ref[...] + p1_ref[...] + g_ref[...]) * dinv_ref[...], 0.0)
    nrm = jnp.sqrt(jnp.sum(h * h, axis=1, keepdims=True))
    o_ref[...] = h / jnp.maximum(nrm, 1e-12)


_last_tc = pl.pallas_call(
    _last_body,
    grid=(_GRID,),
    in_specs=[_row_spec(), _row_spec(), _row_spec(), _col_spec()],
    out_specs=_row_spec(),
    out_shape=jax.ShapeDtypeStruct((NPAD, DIM), jnp.float32),
)


def kernel(x, edge_index, W0, W1, W2):
    src = edge_index[0].astype(jnp.int32)
    dst = edge_index[1].astype(jnp.int32)
    pad_idx = jnp.full((EPAD - N_EDGES,), NPAD - 1, dtype=jnp.int32)
    src_p = jnp.concatenate([src, pad_idx])
    dst_p = jnp.concatenate([dst, pad_idx])
    x_p = jnp.pad(x, ((0, NPAD - N_NODES), (0, 0)))

    ones_ch = jnp.ones((CH,), jnp.float32)
    zeros_row = jnp.zeros((ROWS_PER_TILE,), jnp.float32)
    zeros_blk = jnp.zeros((64, DIM), jnp.float32)

    hist = _degree_sc(dst_p, ones_ch, zeros_row)
    g, dinv = _first_tc(x_p, W0, hist[0][:, None], hist[1][:, None])
    for W in (W1, W2, None):
        p = _scatter_sc(g, src_p, dst_p, zeros_blk)
        if W is None:
            out = _last_tc(p[0], p[1], g, dinv)
        else:
            g = _mid_tc(p[0], p[1], g, dinv, W)
    return out[:N_NODES]
